# baseline (speedup 1.0000x reference)
import functools

import jax
import jax.numpy as jnp
from jax import lax
from jax.experimental import pallas as pl
from jax.experimental.pallas import tpu as pltpu
from jax.experimental.pallas import tpu_sc as plsc

NUM_EMBEDDINGS = 1024
EMBEDDING_DIM = 64
TOKENS = 16 * 32 * 32
BLOCK_TOKENS = 1024
NUM_BLOCKS = TOKENS // BLOCK_TOKENS


def _argmin_block(z_ref, emb_ref, e_sq_ref, idx_ref):
    z = z_ref[...]
    e_sq = e_sq_ref[...]
    zz = z * z
    ones_row = jnp.ones((1, EMBEDDING_DIM), jnp.float32)
    z_sq = jax.lax.dot_general(
        ones_row, zz, (((1,), (1,)), ((), ())),
        precision=jax.lax.Precision.HIGHEST,
        preferred_element_type=jnp.float32)
    ndot = jax.lax.dot_general(
        emb_ref[...], -2.0 * z, (((1,), (1,)), ((), ())),
        preferred_element_type=jnp.float32)
    dist = (z_sq + e_sq) + ndot
    minv = jnp.min(dist, axis=0, keepdims=True)
    iota_f = jax.lax.broadcasted_iota(jnp.int32, dist.shape, 0
                                      ).astype(jnp.float32)
    idx_f = jnp.min(jnp.where(dist == minv, iota_f, float(NUM_EMBEDDINGS)),
                    axis=0)
    idx_ref[0, 0, :] = idx_f.astype(jnp.int32)


def _compute_indices(flat, embedding, e_sq):
    return pl.pallas_call(
        _argmin_block,
        grid=(NUM_BLOCKS,),
        in_specs=[
            pl.BlockSpec((BLOCK_TOKENS, EMBEDDING_DIM), lambda b: (b, 0)),
            pl.BlockSpec((NUM_EMBEDDINGS, EMBEDDING_DIM), lambda b: (0, 0)),
            pl.BlockSpec((NUM_EMBEDDINGS, 1), lambda b: (0, 0)),
        ],
        out_specs=pl.BlockSpec((1, 1, BLOCK_TOKENS), lambda b: (b, 0, 0)),
        out_shape=jax.ShapeDtypeStruct((NUM_BLOCKS, 1, BLOCK_TOKENS),
                                       jnp.int32),
    )(flat, embedding, e_sq)


def _make_sc_gather():
    info = plsc.get_sparse_core_info()
    nc, ns = info.num_cores, info.num_subcores
    nw = nc * ns
    b_per_w = TOKENS // nw
    mesh = plsc.VectorSubcoreMesh(core_axis_name="c", subcore_axis_name="s")

    @functools.partial(
        pl.kernel, mesh=mesh,
        compiler_params=pltpu.CompilerParams(use_tc_tiling_on_sc=False),
        out_type=jax.ShapeDtypeStruct((TOKENS, EMBEDDING_DIM), jnp.float32),
        scratch_types=[
            pltpu.VMEM((b_per_w,), jnp.int32),
            pltpu.VMEM((b_per_w, EMBEDDING_DIM), jnp.float32),
            pltpu.SemaphoreType.DMA,
        ],
    )
    def gather(table_hbm, idx_hbm, out_hbm, idx_v, rows_v, sem):
        wid = lax.axis_index("s") * nc + lax.axis_index("c")
        base = wid * b_per_w
        pltpu.sync_copy(idx_hbm.at[pl.ds(base, b_per_w)], idx_v)
        pltpu.async_copy(table_hbm.at[idx_v], rows_v, sem).wait()
        pltpu.sync_copy(rows_v, out_hbm.at[pl.ds(base, b_per_w)])

    return gather


_sc_gather = _make_sc_gather()


def kernel(hidden_states, embedding):
    flat = hidden_states.reshape(TOKENS, EMBEDDING_DIM)
    e_sq = jnp.sum(embedding ** 2, axis=1)[:, None]
    table = embedding.astype(jnp.bfloat16).astype(jnp.float32)

    idx = _compute_indices(flat, embedding, e_sq)
    idx_flat = idx.reshape(TOKENS)
    quant = _sc_gather(table, idx_flat)

    z_q = quant.reshape(hidden_states.shape)
    B = hidden_states.shape[0]
    min_encoding_indices = idx_flat.reshape(B, TOKENS // B)
    return (z_q, min_encoding_indices)


# speedup vs baseline: 1.2477x; 1.2477x over previous
import functools

import jax
import jax.numpy as jnp
from jax import lax
from jax.experimental import pallas as pl
from jax.experimental.pallas import tpu as pltpu
from jax.experimental.pallas import tpu_sc as plsc

NUM_EMBEDDINGS = 1024
EMBEDDING_DIM = 64
TOKENS = 16 * 32 * 32
BLOCK_TOKENS = 1024
NUM_BLOCKS = TOKENS // BLOCK_TOKENS


def _vq_block(z_ref, emb_ref, z_sq_ref, e_sq_ref, quant_ref, idx_ref):
    z = z_ref[...]
    emb = emb_ref[...]
    z_sq = z_sq_ref[...]
    e_sq = e_sq_ref[...]
    ndot = jax.lax.dot_general(
        emb, -2.0 * z, (((1,), (1,)), ((), ())),
        preferred_element_type=jnp.float32)
    dist = (z_sq + e_sq) + ndot
    minv = jnp.min(dist, axis=0, keepdims=True)
    iota_f = jax.lax.broadcasted_iota(jnp.int32, dist.shape, 0
                                      ).astype(jnp.float32)
    idx_f = jnp.min(jnp.where(dist == minv, iota_f, float(NUM_EMBEDDINGS)),
                    axis=0)
    idx_ref[0, 0, :] = idx_f.astype(jnp.int32)
    onehot_t = (jax.lax.broadcasted_iota(jnp.int32, (NUM_EMBEDDINGS, 1), 0)
                == idx_f.astype(jnp.int32)[None, :]).astype(jnp.float32)
    quant_ref[...] = jax.lax.dot_general(
        onehot_t, emb, (((0,), (0,)), ((), ())),
        preferred_element_type=jnp.float32)


def _vq_tc(flat, embedding, z_sq, e_sq):
    return pl.pallas_call(
        _vq_block,
        grid=(NUM_BLOCKS,),
        in_specs=[
            pl.BlockSpec((BLOCK_TOKENS, EMBEDDING_DIM), lambda b: (b, 0)),
            pl.BlockSpec((NUM_EMBEDDINGS, EMBEDDING_DIM), lambda b: (0, 0)),
            pl.BlockSpec((1, BLOCK_TOKENS), lambda b: (0, b)),
            pl.BlockSpec((NUM_EMBEDDINGS, 1), lambda b: (0, 0)),
        ],
        out_specs=[
            pl.BlockSpec((BLOCK_TOKENS, EMBEDDING_DIM), lambda b: (b, 0)),
            pl.BlockSpec((1, 1, BLOCK_TOKENS), lambda b: (b, 0, 0)),
        ],
        out_shape=[
            jax.ShapeDtypeStruct((TOKENS, EMBEDDING_DIM), jnp.float32),
            jax.ShapeDtypeStruct((NUM_BLOCKS, 1, BLOCK_TOKENS), jnp.int32),
        ],
    )(flat, embedding, z_sq, e_sq)


def _make_sc_gather():
    info = plsc.get_sparse_core_info()
    nc, ns = info.num_cores, info.num_subcores
    nw = nc * ns
    b_per_w = TOKENS // nw
    mesh = plsc.VectorSubcoreMesh(core_axis_name="c", subcore_axis_name="s")

    @functools.partial(
        pl.kernel, mesh=mesh,
        compiler_params=pltpu.CompilerParams(use_tc_tiling_on_sc=False),
        out_type=jax.ShapeDtypeStruct((TOKENS, EMBEDDING_DIM), jnp.float32),
        scratch_types=[
            pltpu.VMEM((b_per_w,), jnp.int32),
            pltpu.VMEM((b_per_w, EMBEDDING_DIM), jnp.float32),
            pltpu.SemaphoreType.DMA,
        ],
    )
    def gather(table_hbm, idx_hbm, out_hbm, idx_v, rows_v, sem):
        wid = lax.axis_index("s") * nc + lax.axis_index("c")
        base = wid * b_per_w
        pltpu.sync_copy(idx_hbm.at[pl.ds(base, b_per_w)], idx_v)
        pltpu.async_copy(table_hbm.at[idx_v], rows_v, sem).wait()
        pltpu.sync_copy(rows_v, out_hbm.at[pl.ds(base, b_per_w)])

    return gather


_sc_gather = _make_sc_gather()


def kernel(hidden_states, embedding):
    flat = hidden_states.reshape(TOKENS, EMBEDDING_DIM)
    z_sq = jnp.sum(flat ** 2, axis=1)[None, :]
    e_sq = jnp.sum(embedding ** 2, axis=1)[:, None]

    quant, idx = _vq_tc(flat, embedding, z_sq, e_sq)
    idx_flat = idx.reshape(TOKENS)

    z_q = quant.reshape(hidden_states.shape)
    B = hidden_states.shape[0]
    min_encoding_indices = idx_flat.reshape(B, TOKENS // B)
    return (z_q, min_encoding_indices)


# speedup vs baseline: 1.2586x; 1.0088x over previous
import functools

import jax
import jax.numpy as jnp
from jax import lax
from jax.experimental import pallas as pl
from jax.experimental.pallas import tpu as pltpu
from jax.experimental.pallas import tpu_sc as plsc

NUM_EMBEDDINGS = 1024
EMBEDDING_DIM = 64
TOKENS = 16 * 32 * 32
BLOCK_TOKENS = 1024
NUM_BLOCKS = TOKENS // BLOCK_TOKENS


def _vq_block(z_ref, emb_ref, z_sq_ref, e_sq_ref, quant_ref, idx_ref):
    z = z_ref[...]
    emb = emb_ref[...]
    z_sq = z_sq_ref[...]
    e_sq = e_sq_ref[...]
    ndot = jax.lax.dot_general(
        emb, -2.0 * z, (((1,), (1,)), ((), ())),
        preferred_element_type=jnp.float32)
    dist = (z_sq + e_sq) + ndot
    minv = jnp.min(dist, axis=0, keepdims=True)
    iota_col = jax.lax.broadcasted_iota(jnp.int32, (NUM_EMBEDDINGS, 1), 0
                                        ).astype(jnp.float32)
    idx_f = jnp.min(jnp.where(dist == minv, iota_col, float(NUM_EMBEDDINGS)),
                    axis=0)
    idx_i = idx_f.astype(jnp.int32)
    idx_ref[0, 0, :] = idx_i
    onehot_t = (jax.lax.broadcasted_iota(jnp.int32, (NUM_EMBEDDINGS, 1), 0)
                == idx_i[None, :]).astype(jnp.bfloat16)
    quant_ref[...] = jax.lax.dot_general(
        onehot_t, emb, (((0,), (0,)), ((), ())),
        preferred_element_type=jnp.float32)


def _vq_tc(flat, embedding, z_sq, e_sq):
    return pl.pallas_call(
        _vq_block,
        grid=(NUM_BLOCKS,),
        compiler_params=pltpu.CompilerParams(
            dimension_semantics=("parallel",)),
        in_specs=[
            pl.BlockSpec((BLOCK_TOKENS, EMBEDDING_DIM), lambda b: (b, 0)),
            pl.BlockSpec((NUM_EMBEDDINGS, EMBEDDING_DIM), lambda b: (0, 0)),
            pl.BlockSpec((1, BLOCK_TOKENS), lambda b: (0, b)),
            pl.BlockSpec((NUM_EMBEDDINGS, 1), lambda b: (0, 0)),
        ],
        out_specs=[
            pl.BlockSpec((BLOCK_TOKENS, EMBEDDING_DIM), lambda b: (b, 0)),
            pl.BlockSpec((1, 1, BLOCK_TOKENS), lambda b: (b, 0, 0)),
        ],
        out_shape=[
            jax.ShapeDtypeStruct((TOKENS, EMBEDDING_DIM), jnp.float32),
            jax.ShapeDtypeStruct((NUM_BLOCKS, 1, BLOCK_TOKENS), jnp.int32),
        ],
    )(flat, embedding, z_sq, e_sq)


def _make_sc_gather():
    info = plsc.get_sparse_core_info()
    nc, ns = info.num_cores, info.num_subcores
    nw = nc * ns
    b_per_w = TOKENS // nw
    mesh = plsc.VectorSubcoreMesh(core_axis_name="c", subcore_axis_name="s")

    @functools.partial(
        pl.kernel, mesh=mesh,
        compiler_params=pltpu.CompilerParams(use_tc_tiling_on_sc=False),
        out_type=jax.ShapeDtypeStruct((TOKENS, EMBEDDING_DIM), jnp.float32),
        scratch_types=[
            pltpu.VMEM((b_per_w,), jnp.int32),
            pltpu.VMEM((b_per_w, EMBEDDING_DIM), jnp.float32),
            pltpu.SemaphoreType.DMA,
        ],
    )
    def gather(table_hbm, idx_hbm, out_hbm, idx_v, rows_v, sem):
        wid = lax.axis_index("s") * nc + lax.axis_index("c")
        base = wid * b_per_w
        pltpu.sync_copy(idx_hbm.at[pl.ds(base, b_per_w)], idx_v)
        pltpu.async_copy(table_hbm.at[idx_v], rows_v, sem).wait()
        pltpu.sync_copy(rows_v, out_hbm.at[pl.ds(base, b_per_w)])

    return gather


_sc_gather = _make_sc_gather()


def kernel(hidden_states, embedding):
    flat = hidden_states.reshape(TOKENS, EMBEDDING_DIM)
    z_sq = jnp.sum(flat ** 2, axis=1)[None, :]
    e_sq = jnp.sum(embedding ** 2, axis=1)[:, None]

    quant, idx = _vq_tc(flat, embedding, z_sq, e_sq)
    idx_flat = idx.reshape(TOKENS)

    z_q = quant.reshape(hidden_states.shape)
    B = hidden_states.shape[0]
    min_encoding_indices = idx_flat.reshape(B, TOKENS // B)
    return (z_q, min_encoding_indices)


# speedup vs baseline: 1.3581x; 1.0791x over previous
import functools

import jax
import jax.numpy as jnp
from jax import lax
from jax.experimental import pallas as pl
from jax.experimental.pallas import tpu as pltpu
from jax.experimental.pallas import tpu_sc as plsc

NUM_EMBEDDINGS = 1024
EMBEDDING_DIM = 64
TOKENS = 16 * 32 * 32
BLOCK_TOKENS = 1024
NUM_BLOCKS = TOKENS // BLOCK_TOKENS


def _vq_block(z_ref, emb_ref, quant_ref, idx_ref):
    z = z_ref[...]
    emb = emb_ref[...]
    e_sq = jnp.sum(emb * emb, axis=1, keepdims=True)
    z_sq = jnp.sum(z * z, axis=1, keepdims=True).T
    ndot = jax.lax.dot_general(
        emb, -2.0 * z, (((1,), (1,)), ((), ())),
        preferred_element_type=jnp.float32)
    dist = (z_sq + e_sq) + ndot
    minv = jnp.min(dist, axis=0, keepdims=True)
    iota_col = jax.lax.broadcasted_iota(jnp.int32, (NUM_EMBEDDINGS, 1), 0
                                        ).astype(jnp.float32)
    idx_f = jnp.min(jnp.where(dist == minv, iota_col, float(NUM_EMBEDDINGS)),
                    axis=0)
    idx_i = idx_f.astype(jnp.int32)
    idx_ref[0, 0, :] = idx_i
    onehot_t = (jax.lax.broadcasted_iota(jnp.int32, (NUM_EMBEDDINGS, 1), 0)
                == idx_i[None, :]).astype(jnp.bfloat16)
    quant_ref[...] = jax.lax.dot_general(
        onehot_t, emb, (((0,), (0,)), ((), ())),
        preferred_element_type=jnp.float32)


def _vq_tc(flat, embedding):
    return pl.pallas_call(
        _vq_block,
        grid=(NUM_BLOCKS,),
        compiler_params=pltpu.CompilerParams(
            dimension_semantics=("parallel",)),
        in_specs=[
            pl.BlockSpec((BLOCK_TOKENS, EMBEDDING_DIM), lambda b: (b, 0)),
            pl.BlockSpec((NUM_EMBEDDINGS, EMBEDDING_DIM), lambda b: (0, 0)),
        ],
        out_specs=[
            pl.BlockSpec((BLOCK_TOKENS, EMBEDDING_DIM), lambda b: (b, 0)),
            pl.BlockSpec((1, 1, BLOCK_TOKENS), lambda b: (b, 0, 0)),
        ],
        out_shape=[
            jax.ShapeDtypeStruct((TOKENS, EMBEDDING_DIM), jnp.float32),
            jax.ShapeDtypeStruct((NUM_BLOCKS, 1, BLOCK_TOKENS), jnp.int32),
        ],
    )(flat, embedding)


def _make_sc_gather():
    info = plsc.get_sparse_core_info()
    nc, ns = info.num_cores, info.num_subcores
    nw = nc * ns
    b_per_w = TOKENS // nw
    mesh = plsc.VectorSubcoreMesh(core_axis_name="c", subcore_axis_name="s")

    @functools.partial(
        pl.kernel, mesh=mesh,
        compiler_params=pltpu.CompilerParams(use_tc_tiling_on_sc=False),
        out_type=jax.ShapeDtypeStruct((TOKENS, EMBEDDING_DIM), jnp.float32),
        scratch_types=[
            pltpu.VMEM((b_per_w,), jnp.int32),
            pltpu.VMEM((b_per_w, EMBEDDING_DIM), jnp.float32),
            pltpu.SemaphoreType.DMA,
        ],
    )
    def gather(table_hbm, idx_hbm, out_hbm, idx_v, rows_v, sem):
        wid = lax.axis_index("s") * nc + lax.axis_index("c")
        base = wid * b_per_w
        pltpu.sync_copy(idx_hbm.at[pl.ds(base, b_per_w)], idx_v)
        pltpu.async_copy(table_hbm.at[idx_v], rows_v, sem).wait()
        pltpu.sync_copy(rows_v, out_hbm.at[pl.ds(base, b_per_w)])

    return gather


_sc_gather = _make_sc_gather()


def kernel(hidden_states, embedding):
    flat = hidden_states.reshape(TOKENS, EMBEDDING_DIM)
    quant, idx = _vq_tc(flat, embedding)
    idx_flat = idx.reshape(TOKENS)

    z_q = quant.reshape(hidden_states.shape)
    B = hidden_states.shape[0]
    min_encoding_indices = idx_flat.reshape(B, TOKENS // B)
    return (z_q, min_encoding_indices)


# speedup vs baseline: 1.7367x; 1.2788x over previous
import functools

import jax
import jax.numpy as jnp
from jax import lax
from jax.experimental import pallas as pl
from jax.experimental.pallas import tpu as pltpu
from jax.experimental.pallas import tpu_sc as plsc

NUM_EMBEDDINGS = 1024
EMBEDDING_DIM = 64
TOKENS = 16 * 32 * 32
BLOCK_TOKENS = 2048
NUM_BLOCKS = TOKENS // BLOCK_TOKENS


def _vq_block(z_ref, emb_ref, quant_ref, idx_ref):
    z = z_ref[...]
    emb = emb_ref[...]
    e_sq = jnp.sum(emb * emb, axis=1, keepdims=True)
    z_sq = jnp.sum(z * z, axis=1, keepdims=True).T
    ndot = jax.lax.dot_general(
        emb, -2.0 * z, (((1,), (1,)), ((), ())),
        preferred_element_type=jnp.float32)
    dist = (z_sq + e_sq) + ndot
    minv = jnp.min(dist, axis=0, keepdims=True)
    iota_col = jax.lax.broadcasted_iota(jnp.int32, (NUM_EMBEDDINGS, 1), 0
                                        ).astype(jnp.float32)
    idx_f = jnp.min(jnp.where(dist == minv, iota_col, float(NUM_EMBEDDINGS)),
                    axis=0)
    idx_i = idx_f.astype(jnp.int32)
    idx_ref[0, 0, :] = idx_i
    onehot_t = (jax.lax.broadcasted_iota(jnp.int32, (NUM_EMBEDDINGS, 1), 0)
                == idx_i[None, :]).astype(jnp.bfloat16)
    quant_ref[...] = jax.lax.dot_general(
        onehot_t, emb, (((0,), (0,)), ((), ())),
        preferred_element_type=jnp.float32)


def _vq_tc(flat, embedding):
    return pl.pallas_call(
        _vq_block,
        grid=(NUM_BLOCKS,),
        compiler_params=pltpu.CompilerParams(
            dimension_semantics=("parallel",)),
        in_specs=[
            pl.BlockSpec((BLOCK_TOKENS, EMBEDDING_DIM), lambda b: (b, 0)),
            pl.BlockSpec((NUM_EMBEDDINGS, EMBEDDING_DIM), lambda b: (0, 0)),
        ],
        out_specs=[
            pl.BlockSpec((BLOCK_TOKENS, EMBEDDING_DIM), lambda b: (b, 0)),
            pl.BlockSpec((1, 1, BLOCK_TOKENS), lambda b: (b, 0, 0)),
        ],
        out_shape=[
            jax.ShapeDtypeStruct((TOKENS, EMBEDDING_DIM), jnp.float32),
            jax.ShapeDtypeStruct((NUM_BLOCKS, 1, BLOCK_TOKENS), jnp.int32),
        ],
    )(flat, embedding)


def _make_sc_gather():
    info = plsc.get_sparse_core_info()
    nc, ns = info.num_cores, info.num_subcores
    nw = nc * ns
    b_per_w = TOKENS // nw
    mesh = plsc.VectorSubcoreMesh(core_axis_name="c", subcore_axis_name="s")

    @functools.partial(
        pl.kernel, mesh=mesh,
        compiler_params=pltpu.CompilerParams(use_tc_tiling_on_sc=False),
        out_type=jax.ShapeDtypeStruct((TOKENS, EMBEDDING_DIM), jnp.float32),
        scratch_types=[
            pltpu.VMEM((b_per_w,), jnp.int32),
            pltpu.VMEM((b_per_w, EMBEDDING_DIM), jnp.float32),
            pltpu.SemaphoreType.DMA,
        ],
    )
    def gather(table_hbm, idx_hbm, out_hbm, idx_v, rows_v, sem):
        wid = lax.axis_index("s") * nc + lax.axis_index("c")
        base = wid * b_per_w
        pltpu.sync_copy(idx_hbm.at[pl.ds(base, b_per_w)], idx_v)
        pltpu.async_copy(table_hbm.at[idx_v], rows_v, sem).wait()
        pltpu.sync_copy(rows_v, out_hbm.at[pl.ds(base, b_per_w)])

    return gather


_sc_gather = _make_sc_gather()


def kernel(hidden_states, embedding):
    flat = hidden_states.reshape(TOKENS, EMBEDDING_DIM)
    quant, idx = _vq_tc(flat, embedding)
    idx_flat = idx.reshape(TOKENS)

    z_q = quant.reshape(hidden_states.shape)
    B = hidden_states.shape[0]
    min_encoding_indices = idx_flat.reshape(B, TOKENS // B)
    return (z_q, min_encoding_indices)
